# baseline (speedup 1.0000x reference)
import functools

import jax
import jax.numpy as jnp
from jax import lax
from jax.experimental import pallas as pl
from jax.experimental.pallas import tpu as pltpu
from jax.experimental.pallas import tpu_sc as plsc

C = 96
P = 5
B_ = 2
H = 224
W = 224
NH = 45
NW_G = 45
L = NH * NW_G
D = C * P * P
DP = 2432
NROWS = B_ * 3 * L
NROWS_PAD = 12288
R_BLK = 16
N_BLK = H // R_BLK
WPAD = W + 2
FLAT_IN = (R_BLK + 3) * WPAD
FLAT_IN_PAD = 4296
FLAT_OUT = R_BLK * WPAD


def _sc_gather_scale(table, idx, vals):
    info = plsc.get_sparse_core_info()
    nc, ns = info.num_cores, info.num_subcores
    nw = nc * ns
    pw = NROWS_PAD // nw
    ch = 16
    nchunk = pw // ch
    mesh = plsc.VectorSubcoreMesh(core_axis_name="c", subcore_axis_name="s")

    @functools.partial(
        pl.kernel, mesh=mesh,
        out_type=jax.ShapeDtypeStruct((NROWS_PAD, DP), jnp.float32),
        scratch_types=[
            pltpu.VMEM((pw,), jnp.int32),
            pltpu.VMEM((pw, 16), jnp.float32),
            pltpu.VMEM((ch, DP), jnp.float32),
            pltpu.SemaphoreType.DMA,
        ],
    )
    def k(table_hbm, idx_hbm, vals_hbm, out_hbm, idx_v, vals_v, rows_v, sem):
        wid = lax.axis_index("s") * nc + lax.axis_index("c")
        base = wid * pw
        pltpu.sync_copy(idx_hbm.at[pl.ds(base, pw)], idx_v)
        pltpu.sync_copy(vals_hbm.at[pl.ds(base, pw)], vals_v)

        def chunk_body(ci, carry):
            row0 = ci * ch
            pltpu.async_copy(
                table_hbm.at[idx_v.at[pl.ds(row0, ch)]], rows_v, sem).wait()

            def row_body(r, c2):
                vv = vals_v[row0 + r]
                for j in range(DP // 16):
                    sl = pl.ds(j * 16, 16)
                    rows_v[r, sl] = rows_v[r, sl] * vv
                return c2

            lax.fori_loop(0, ch, row_body, 0)
            pltpu.sync_copy(rows_v, out_hbm.at[pl.ds(base + row0, ch)])
            return carry

        lax.fori_loop(0, nchunk, chunk_body, 0)

    return k(table, idx, vals)


def _wnorm_body(wv_ref, wg_ref, o_ref):
    w = wv_ref[...]
    n2 = jnp.sum(w * w, axis=0, keepdims=True)
    o_ref[...] = w * (wg_ref[...] * lax.rsqrt(n2))


def _conv_body(x_ref, w_ref, b_ref, o_ref):
    acc = jnp.zeros((FLAT_OUT, C), jnp.float32)
    for dy in range(3):
        for dx in range(3):
            off = dy * WPAD + dx
            a = x_ref[0, 0, pl.ds(off, FLAT_OUT), :]
            acc = acc + jnp.dot(a, w_ref[dy, dx],
                                preferred_element_type=jnp.float32)
    o_ref[0, 0] = acc + b_ref[0]


def kernel(img, corr_index, corr_values, w_v, w_g, bias):
    f32 = jnp.float32
    xpad = jnp.pad(img, ((0, 0), (0, 0), (0, 1), (0, 1)))
    table = (xpad.reshape(B_, C, NH, P, NW_G, P)
             .transpose(0, 2, 4, 1, 3, 5)
             .reshape(B_ * L, D))
    table = jnp.pad(table, ((0, 0), (0, DP - D)))
    ci = corr_index.transpose(1, 2, 0)
    idx_flat = (ci + (jnp.arange(B_, dtype=jnp.int32) * L)[:, None, None]
                ).reshape(-1)
    idx_flat = jnp.concatenate(
        [idx_flat, jnp.zeros((NROWS_PAD - NROWS,), jnp.int32)])
    vals_flat = corr_values.transpose(1, 2, 0).reshape(-1)
    vals_flat = jnp.concatenate(
        [vals_flat, jnp.zeros((NROWS_PAD - NROWS,), f32)])
    vals2 = jnp.broadcast_to(vals_flat[:, None], (NROWS_PAD, 16))

    g = _sc_gather_scale(table, idx_flat, vals2)

    gv = (g[:NROWS, :D].reshape(B_, 3, NH, NW_G, C, P, P)
          .transpose(0, 2, 5, 3, 6, 1, 4)
          .reshape(B_, NH * P, NW_G * P, 3 * C)[:, :H, :W, :])
    img_nhwc = jnp.transpose(img, (0, 2, 3, 1))
    fusion = jnp.concatenate([img_nhwc, gv], axis=3)
    fusion = jnp.pad(fusion, ((0, 0), (1, 2), (1, 1), (0, 0)))
    fusion_ov = jnp.stack(
        [fusion[:, R_BLK * i:R_BLK * i + R_BLK + 3] for i in range(N_BLK)],
        axis=1).reshape(B_, N_BLK, FLAT_IN, 4 * C)
    fusion_ov = jnp.pad(
        fusion_ov, ((0, 0), (0, 0), (0, FLAT_IN_PAD - FLAT_IN), (0, 0)))

    w_vt = jnp.transpose(w_v.reshape(C, 4 * C * 9), (1, 0))
    wsc = pl.pallas_call(
        _wnorm_body,
        out_shape=jax.ShapeDtypeStruct((4 * C * 9, C), f32),
    )(w_vt, w_g.reshape(1, C))
    wtaps = wsc.reshape(4 * C, 3, 3, C).transpose(1, 2, 0, 3)

    out = pl.pallas_call(
        _conv_body,
        grid=(B_, N_BLK),
        in_specs=[
            pl.BlockSpec((1, 1, FLAT_IN_PAD, 4 * C), lambda b, i: (b, i, 0, 0)),
            pl.BlockSpec((3, 3, 4 * C, C), lambda b, i: (0, 0, 0, 0)),
            pl.BlockSpec((1, C), lambda b, i: (0, 0)),
        ],
        out_specs=pl.BlockSpec((1, 1, FLAT_OUT, C), lambda b, i: (b, i, 0, 0)),
        out_shape=jax.ShapeDtypeStruct((B_, N_BLK, FLAT_OUT, C), f32),
    )(fusion_ov, wtaps, bias.reshape(1, C))

    out = (out.reshape(B_, N_BLK, R_BLK, WPAD, C)[:, :, :, :W, :]
           .reshape(B_, H, W, C).transpose(0, 3, 1, 2))
    return out


# speedup vs baseline: 3.0858x; 3.0858x over previous
import functools

import jax
import jax.numpy as jnp
import numpy as np
from jax import lax
from jax.experimental import pallas as pl
from jax.experimental.pallas import tpu as pltpu
from jax.experimental.pallas import tpu_sc as plsc

C = 96
P = 5
B_ = 2
H = 224
W = 224
NH = 45
L = NH * NH
D = C * P * P
DP = 2432
ZROW = B_ * L

HPAD = H + 3
WPAD = W + 2
BSTRIDE = HPAD * WPAD + 2
NPIX = B_ * BSTRIDE
NWORK = 32
F_ROWS = NPIX + 40

CHP = 4
CHPIX = CHP * 25 + 4
NCH = 33
PPW = CHP * NCH
GP_TOTAL = NWORK * PPW
N_BORDER_P = 91

R_BLK = 16
N_BLK = H // R_BLK
FLAT_IN = (R_BLK + 3) * WPAD + 2
FLAT_OUT = R_BLK * WPAD


def _pix_indices():
    gp = np.arange(GP_TOTAL)
    wk = gp // PPW
    trash = NPIX + wk
    pix = np.repeat(trash[:, None], 25, axis=1).astype(np.int64)
    g = np.arange(B_ * L)
    b, k = g // L, g % L
    i, j = k // NH, k % NH
    u = np.arange(P)[:, None]
    v = np.arange(P)[None, :]
    Y = (P * i)[:, None, None] + u[None]
    X = (P * j)[:, None, None] + v[None]
    t = b[:, None, None] * BSTRIDE + (Y + 1) * WPAD + (X + 1)
    t = np.where((Y < H) & (X < W), t, (NPIX + wk[:B_ * L])[:, None, None])
    pix[:B_ * L] = t.reshape(B_ * L, 25)
    yy = np.arange(HPAD)[:, None]
    xx = np.arange(WPAD)[None, :]
    m = (yy == 0) | (yy >= HPAD - 2) | (xx == 0) | (xx == WPAD - 1)
    bidx = np.concatenate(
        [bb * BSTRIDE + np.nonzero(m.reshape(-1))[0] for bb in range(B_)])
    full = np.repeat(trash[B_ * L:B_ * L + N_BORDER_P], 25).astype(np.int64)
    full[:bidx.size] = bidx
    pix[B_ * L:B_ * L + N_BORDER_P] = full.reshape(N_BORDER_P, 25)
    pixw = pix.reshape(NWORK, NCH, CHP * 25)
    tr = np.broadcast_to((NPIX + np.arange(NWORK))[:, None, None],
                         (NWORK, NCH, CHPIX - CHP * 25))
    return jnp.asarray(
        np.concatenate([pixw, tr], axis=2).astype(np.int32))


_PIX = None


def _sc_compose(table, gidx, valse):
    global _PIX
    if _PIX is None:
        _PIX = _pix_indices()
    mesh = plsc.VectorSubcoreMesh(core_axis_name="c", subcore_axis_name="s")
    info = plsc.get_sparse_core_info()
    nc = info.num_cores

    @functools.partial(
        pl.kernel, mesh=mesh,
        out_type=jax.ShapeDtypeStruct((F_ROWS, 4 * C), jnp.float32),
        scratch_types=[
            pltpu.VMEM((NCH, 16), jnp.int32),
            pltpu.VMEM((NCH, CHPIX), jnp.int32),
            pltpu.VMEM((PPW * 48,), jnp.float32),
            pltpu.VMEM((4 * CHP, DP), jnp.float32),
            pltpu.VMEM((CHPIX, 4 * C), jnp.float32),
            pltpu.SemaphoreType.DMA,
            pltpu.SemaphoreType.DMA,
        ],
    )
    def k(table_h, gidx_h, pix_h, valse_h, f_h,
          gidx_v, pix_v, vals_v, gbuf, dest, sem_g, sem_s):
        wid = lax.axis_index("s") * nc + lax.axis_index("c")
        pltpu.sync_copy(gidx_h.at[wid], gidx_v)
        pltpu.sync_copy(pix_h.at[wid], pix_v)
        pltpu.sync_copy(valse_h.at[wid], vals_v)

        def chunk_body(ci, carry):
            pltpu.async_copy(table_h.at[gidx_v.at[ci]], gbuf, sem_g).wait()

            def patch_body(p, c2):
                r0 = p * 4
                d0 = p * 25
                vb = (ci * CHP + p) * 48
                vv0 = vals_v[pl.ds(vb, 16)]
                vv1 = vals_v[pl.ds(vb + 16, 16)]
                vv2 = vals_v[pl.ds(vb + 32, 16)]

                def seg_body(seg, c3):
                    col = seg * C
                    for cc in range(C // 16):
                        s = pl.ds(col + cc * 16, 16)
                        o = cc * 16
                        dest[d0 + seg, pl.ds(o, 16)] = gbuf[r0, s]
                        dest[d0 + seg, pl.ds(C + o, 16)] = gbuf[r0 + 1, s] * vv0
                        dest[d0 + seg, pl.ds(2 * C + o, 16)] = gbuf[r0 + 2, s] * vv1
                        dest[d0 + seg, pl.ds(3 * C + o, 16)] = gbuf[r0 + 3, s] * vv2
                    return c3

                lax.fori_loop(0, 25, seg_body, 0)
                return c2

            lax.fori_loop(0, CHP, patch_body, 0)
            pltpu.async_copy(dest, f_h.at[pix_v.at[ci]], sem_s).wait()
            return carry

        lax.fori_loop(0, NCH, chunk_body, 0)

    return k(table, gidx, _PIX, valse)


def _wnorm_body(wv_ref, wg_ref, o_ref):
    w = wv_ref[...]
    n2 = jnp.sum(w * w, axis=0, keepdims=True)
    o_ref[...] = w * (wg_ref[...] * lax.rsqrt(n2))


def _conv_body(f_ref, w_ref, b_ref, o_ref, buf, sem):
    b = pl.program_id(0)
    i = pl.program_id(1)
    s0 = b * BSTRIDE + R_BLK * i * WPAD
    cp = pltpu.make_async_copy(f_ref.at[pl.ds(s0, FLAT_IN)], buf, sem)
    cp.start()
    cp.wait()
    acc = jnp.zeros((FLAT_OUT, C), jnp.float32)
    for dy in range(3):
        for dx in range(3):
            off = dy * WPAD + dx
            a = buf[pl.ds(off, FLAT_OUT), :]
            acc = acc + jnp.dot(a, w_ref[dy, dx],
                                preferred_element_type=jnp.float32)
    o_ref[0, 0] = acc + b_ref[0]


def kernel(img, corr_index, corr_values, w_v, w_g, bias):
    f32 = jnp.float32
    xpad = jnp.pad(img, ((0, 0), (0, 0), (0, 1), (0, 1)))
    table = (xpad.reshape(B_, C, NH, P, NH, P)
             .transpose(0, 2, 4, 3, 5, 1)
             .reshape(B_ * L, D))
    table = jnp.pad(table, ((0, 1), (0, DP - D)))

    nbr = (corr_index.transpose(1, 0, 2)
           + (jnp.arange(B_, dtype=jnp.int32) * L)[:, None, None]
           ).reshape(B_ * L, 3)
    gidx = jnp.concatenate(
        [jnp.arange(B_ * L, dtype=jnp.int32)[:, None], nbr], axis=1)
    gidx = jnp.concatenate(
        [gidx, jnp.full((GP_TOTAL - B_ * L, 4), ZROW, jnp.int32)])
    gidx = gidx.reshape(NWORK, NCH, 16)

    vals = corr_values.transpose(1, 0, 2).reshape(B_ * L, 3)
    vals = jnp.concatenate([vals, jnp.zeros((GP_TOTAL - B_ * L, 3), f32)])
    valse = jnp.broadcast_to(vals[:, :, None], (GP_TOTAL, 3, 16))
    valse = valse.reshape(NWORK, PPW * 48)

    f_flat = _sc_compose(table, gidx, valse)

    w_vt = jnp.transpose(w_v.reshape(C, 4 * C * 9), (1, 0))
    wsc = pl.pallas_call(
        _wnorm_body,
        out_shape=jax.ShapeDtypeStruct((4 * C * 9, C), f32),
    )(w_vt, w_g.reshape(1, C))
    wtaps = wsc.reshape(4 * C, 3, 3, C).transpose(1, 2, 0, 3)

    out = pl.pallas_call(
        _conv_body,
        grid=(B_, N_BLK),
        in_specs=[
            pl.BlockSpec(memory_space=pl.ANY),
            pl.BlockSpec((3, 3, 4 * C, C), lambda b, i: (0, 0, 0, 0)),
            pl.BlockSpec((1, C), lambda b, i: (0, 0)),
        ],
        out_specs=pl.BlockSpec((1, 1, FLAT_OUT, C), lambda b, i: (b, i, 0, 0)),
        out_shape=jax.ShapeDtypeStruct((B_, N_BLK, FLAT_OUT, C), f32),
        scratch_shapes=[
            pltpu.VMEM((FLAT_IN, 4 * C), f32),
            pltpu.SemaphoreType.DMA,
        ],
    )(f_flat, wtaps, bias.reshape(1, C))

    out = (out.reshape(B_, N_BLK, R_BLK, WPAD, C)[:, :, :, :W, :]
           .reshape(B_, H, W, C).transpose(0, 3, 1, 2))
    return out


# speedup vs baseline: 3.3213x; 1.0763x over previous
import functools

import jax
import jax.numpy as jnp
import numpy as np
from jax import lax
from jax.experimental import pallas as pl
from jax.experimental.pallas import tpu as pltpu
from jax.experimental.pallas import tpu_sc as plsc

C = 96
P = 5
B_ = 2
H = 224
W = 224
NH = 45
L = NH * NH
D = C * P * P
DP = 2432
ZROW = B_ * L

HPAD = H + 3
WPAD = W + 2
BSTRIDE = HPAD * WPAD + 2
NPIX = B_ * BSTRIDE
NWORK = 32
F_ROWS = NPIX + 40

CHP = 4
CHPIX = CHP * 25 + 4
NCH = 33
PPW = CHP * NCH
GP_TOTAL = NWORK * PPW
N_BORDER_P = 91

R_BLK = 16
N_BLK = H // R_BLK
FLAT_IN = (R_BLK + 3) * WPAD + 2
FLAT_OUT = R_BLK * WPAD


def _pix_indices():
    gp = np.arange(GP_TOTAL)
    wk = gp // PPW
    trash = NPIX + wk
    pix = np.repeat(trash[:, None], 25, axis=1).astype(np.int64)
    g = np.arange(B_ * L)
    b, k = g // L, g % L
    i, j = k // NH, k % NH
    u = np.arange(P)[:, None]
    v = np.arange(P)[None, :]
    Y = (P * i)[:, None, None] + u[None]
    X = (P * j)[:, None, None] + v[None]
    t = b[:, None, None] * BSTRIDE + (Y + 1) * WPAD + (X + 1)
    t = np.where((Y < H) & (X < W), t, (NPIX + wk[:B_ * L])[:, None, None])
    pix[:B_ * L] = t.reshape(B_ * L, 25)
    yy = np.arange(HPAD)[:, None]
    xx = np.arange(WPAD)[None, :]
    m = (yy == 0) | (yy >= HPAD - 2) | (xx == 0) | (xx == WPAD - 1)
    bidx = np.concatenate(
        [bb * BSTRIDE + np.nonzero(m.reshape(-1))[0] for bb in range(B_)])
    full = np.repeat(trash[B_ * L:B_ * L + N_BORDER_P], 25).astype(np.int64)
    full[:bidx.size] = bidx
    pix[B_ * L:B_ * L + N_BORDER_P] = full.reshape(N_BORDER_P, 25)
    pixw = pix.reshape(NWORK, NCH, CHP * 25)
    tr = np.broadcast_to((NPIX + np.arange(NWORK))[:, None, None],
                         (NWORK, NCH, CHPIX - CHP * 25))
    return jnp.asarray(
        np.concatenate([pixw, tr], axis=2).astype(np.int32))


_PIX = None


def _sc_compose(table, gidx, valse):
    global _PIX
    if _PIX is None:
        _PIX = _pix_indices()
    mesh = plsc.VectorSubcoreMesh(core_axis_name="c", subcore_axis_name="s")
    info = plsc.get_sparse_core_info()
    nc = info.num_cores

    @functools.partial(
        pl.kernel, mesh=mesh,
        out_type=jax.ShapeDtypeStruct((F_ROWS, 4 * C), jnp.float32),
        scratch_types=[
            pltpu.VMEM((NCH, 16), jnp.int32),
            pltpu.VMEM((NCH, CHPIX), jnp.int32),
            pltpu.VMEM((PPW * 48,), jnp.float32),
            pltpu.VMEM((4 * CHP, DP), jnp.float32),
            pltpu.VMEM((CHPIX, 4 * C), jnp.float32),
            pltpu.SemaphoreType.DMA,
            pltpu.SemaphoreType.DMA,
        ],
    )
    def k(table_h, gidx_h, pix_h, valse_h, f_h,
          gidx_v, pix_v, vals_v, gbuf, dest, sem_g, sem_s):
        wid = lax.axis_index("s") * nc + lax.axis_index("c")
        pltpu.sync_copy(gidx_h.at[wid], gidx_v)
        pltpu.sync_copy(pix_h.at[wid], pix_v)
        pltpu.sync_copy(valse_h.at[wid], vals_v)

        def chunk_body(ci, carry):
            pltpu.async_copy(table_h.at[gidx_v.at[ci]], gbuf, sem_g).wait()

            def patch_body(p, c2):
                r0 = p * 4
                d0 = p * 25
                vb = (ci * CHP + p) * 48
                vv0 = vals_v[pl.ds(vb, 16)]
                vv1 = vals_v[pl.ds(vb + 16, 16)]
                vv2 = vals_v[pl.ds(vb + 32, 16)]

                def seg_body(seg, c3):
                    col = seg * C
                    for cc in range(C // 16):
                        s = pl.ds(col + cc * 16, 16)
                        o = cc * 16
                        dest[d0 + seg, pl.ds(o, 16)] = gbuf[r0, s]
                        dest[d0 + seg, pl.ds(C + o, 16)] = gbuf[r0 + 1, s] * vv0
                        dest[d0 + seg, pl.ds(2 * C + o, 16)] = gbuf[r0 + 2, s] * vv1
                        dest[d0 + seg, pl.ds(3 * C + o, 16)] = gbuf[r0 + 3, s] * vv2
                    return c3

                lax.fori_loop(0, 25, seg_body, 0)
                return c2

            lax.fori_loop(0, CHP, patch_body, 0)
            pltpu.async_copy(dest, f_h.at[pix_v.at[ci]], sem_s).wait()
            return carry

        lax.fori_loop(0, NCH, chunk_body, 0)

    return k(table, gidx, _PIX, valse)


def _wnorm_body(wv_ref, wg_ref, o_ref):
    w = wv_ref[...]
    n2 = jnp.sum(w * w, axis=0, keepdims=True)
    o_ref[...] = w * (wg_ref[...] * lax.rsqrt(n2))


def _slab_copy(f_ref, buf, sem, step, slot):
    b2 = step // N_BLK
    i2 = step % N_BLK
    s0 = b2 * BSTRIDE + R_BLK * i2 * WPAD
    return pltpu.make_async_copy(
        f_ref.at[pl.ds(s0, FLAT_IN)], buf.at[slot], sem.at[slot])


def _conv_body(f_ref, w_ref, b_ref, o_ref, buf, sem):
    step = pl.program_id(0) * N_BLK + pl.program_id(1)
    slot = lax.rem(step, 2)

    @pl.when(step == 0)
    def _():
        _slab_copy(f_ref, buf, sem, 0, 0).start()

    @pl.when(step + 1 < B_ * N_BLK)
    def _():
        _slab_copy(f_ref, buf, sem, step + 1, lax.rem(step + 1, 2)).start()

    _slab_copy(f_ref, buf, sem, step, slot).wait()
    acc = jnp.zeros((FLAT_OUT, C), jnp.float32)
    for dy in range(3):
        for dx in range(3):
            off = dy * WPAD + dx
            a = buf[slot, pl.ds(off, FLAT_OUT), :]
            acc = acc + jnp.dot(a, w_ref[dy, dx],
                                preferred_element_type=jnp.float32)
    o_ref[0, 0] = acc + b_ref[0]


def kernel(img, corr_index, corr_values, w_v, w_g, bias):
    f32 = jnp.float32
    xpad = jnp.pad(img, ((0, 0), (0, 0), (0, 1), (0, 1)))
    table = (xpad.reshape(B_, C, NH, P, NH, P)
             .transpose(0, 2, 4, 3, 5, 1)
             .reshape(B_ * L, D))
    table = jnp.pad(table, ((0, 1), (0, DP - D)))

    nbr = (corr_index.transpose(1, 0, 2)
           + (jnp.arange(B_, dtype=jnp.int32) * L)[:, None, None]
           ).reshape(B_ * L, 3)
    gidx = jnp.concatenate(
        [jnp.arange(B_ * L, dtype=jnp.int32)[:, None], nbr], axis=1)
    gidx = jnp.concatenate(
        [gidx, jnp.full((GP_TOTAL - B_ * L, 4), ZROW, jnp.int32)])
    gidx = gidx.reshape(NWORK, NCH, 16)

    vals = corr_values.transpose(1, 0, 2).reshape(B_ * L, 3)
    vals = jnp.concatenate([vals, jnp.zeros((GP_TOTAL - B_ * L, 3), f32)])
    valse = jnp.broadcast_to(vals[:, :, None], (GP_TOTAL, 3, 16))
    valse = valse.reshape(NWORK, PPW * 48)

    f_flat = _sc_compose(table, gidx, valse)

    w_vt = jnp.transpose(w_v.reshape(C, 4 * C * 9), (1, 0))
    wsc = pl.pallas_call(
        _wnorm_body,
        out_shape=jax.ShapeDtypeStruct((4 * C * 9, C), f32),
    )(w_vt, w_g.reshape(1, C))
    wtaps = wsc.reshape(4 * C, 3, 3, C).transpose(1, 2, 0, 3)

    out = pl.pallas_call(
        _conv_body,
        grid=(B_, N_BLK),
        in_specs=[
            pl.BlockSpec(memory_space=pl.ANY),
            pl.BlockSpec((3, 3, 4 * C, C), lambda b, i: (0, 0, 0, 0)),
            pl.BlockSpec((1, C), lambda b, i: (0, 0)),
        ],
        out_specs=pl.BlockSpec((1, 1, FLAT_OUT, C), lambda b, i: (b, i, 0, 0)),
        out_shape=jax.ShapeDtypeStruct((B_, N_BLK, FLAT_OUT, C), f32),
        scratch_shapes=[
            pltpu.VMEM((2, FLAT_IN, 4 * C), f32),
            pltpu.SemaphoreType.DMA((2,)),
        ],
    )(f_flat, wtaps, bias.reshape(1, C))

    out = (out.reshape(B_, N_BLK, R_BLK, WPAD, C)[:, :, :, :W, :]
           .reshape(B_, H, W, C).transpose(0, 3, 1, 2))
    return out


# speedup vs baseline: 3.7161x; 1.1189x over previous
import functools

import jax
import jax.numpy as jnp
import numpy as np
from jax import lax
from jax.experimental import pallas as pl
from jax.experimental.pallas import tpu as pltpu
from jax.experimental.pallas import tpu_sc as plsc

C = 96
P = 5
B_ = 2
H = 224
W = 224
NH = 45
L = NH * NH
D = C * P * P
DP = 2432
ZROW = B_ * L

HPAD = H + 3
WPAD = W + 2
BSTRIDE = HPAD * WPAD + 2
NPIX = B_ * BSTRIDE
NWORK = 32
F_ROWS = NPIX + 40

CHP = 2
CHPIX = CHP * 25 + 6
NCH = 66
PPW = CHP * NCH
GP_TOTAL = NWORK * PPW
N_BORDER_P = 91

R_BLK = 16
N_BLK = H // R_BLK
FLAT_IN = (R_BLK + 3) * WPAD + 2
FLAT_OUT = R_BLK * WPAD


def _pix_indices():
    gp = np.arange(GP_TOTAL)
    wk = gp // PPW
    trash = NPIX + wk
    pix = np.repeat(trash[:, None], 25, axis=1).astype(np.int64)
    g = np.arange(B_ * L)
    b, k = g // L, g % L
    i, j = k // NH, k % NH
    u = np.arange(P)[:, None]
    v = np.arange(P)[None, :]
    Y = (P * i)[:, None, None] + u[None]
    X = (P * j)[:, None, None] + v[None]
    t = b[:, None, None] * BSTRIDE + (Y + 1) * WPAD + (X + 1)
    t = np.where((Y < H) & (X < W), t, (NPIX + wk[:B_ * L])[:, None, None])
    pix[:B_ * L] = t.reshape(B_ * L, 25)
    yy = np.arange(HPAD)[:, None]
    xx = np.arange(WPAD)[None, :]
    m = (yy == 0) | (yy >= HPAD - 2) | (xx == 0) | (xx == WPAD - 1)
    bidx = np.concatenate(
        [bb * BSTRIDE + np.nonzero(m.reshape(-1))[0] for bb in range(B_)])
    full = np.repeat(trash[B_ * L:B_ * L + N_BORDER_P], 25).astype(np.int64)
    full[:bidx.size] = bidx
    pix[B_ * L:B_ * L + N_BORDER_P] = full.reshape(N_BORDER_P, 25)
    pixw = pix.reshape(NWORK, NCH, CHP * 25)
    tr = np.broadcast_to((NPIX + np.arange(NWORK))[:, None, None],
                         (NWORK, NCH, CHPIX - CHP * 25))
    return jnp.asarray(
        np.concatenate([pixw, tr], axis=2).astype(np.int32))


_PIX = None


def _sc_compose(table, gidx, valse):
    global _PIX
    if _PIX is None:
        _PIX = _pix_indices()
    mesh = plsc.VectorSubcoreMesh(core_axis_name="c", subcore_axis_name="s")
    info = plsc.get_sparse_core_info()
    nc = info.num_cores

    @functools.partial(
        pl.kernel, mesh=mesh,
        out_type=jax.ShapeDtypeStruct((F_ROWS, 4 * C), jnp.float32),
        scratch_types=[
            pltpu.VMEM((NCH, 4 * CHP), jnp.int32),
            pltpu.VMEM((NCH, CHPIX), jnp.int32),
            pltpu.VMEM((PPW * 48,), jnp.float32),
            pltpu.VMEM((4 * CHP, DP), jnp.float32),
            pltpu.VMEM((4 * CHP, DP), jnp.float32),
            pltpu.VMEM((CHPIX, 4 * C), jnp.float32),
            pltpu.VMEM((CHPIX, 4 * C), jnp.float32),
            pltpu.SemaphoreType.DMA,
            pltpu.SemaphoreType.DMA,
            pltpu.SemaphoreType.DMA,
            pltpu.SemaphoreType.DMA,
        ],
    )
    def k(table_h, gidx_h, pix_h, valse_h, f_h,
          gidx_v, pix_v, vals_v, g0, g1, d0, d1,
          sem_g0, sem_g1, sem_s0, sem_s1):
        wid = lax.axis_index("s") * nc + lax.axis_index("c")
        pltpu.sync_copy(gidx_h.at[wid], gidx_v)
        pltpu.sync_copy(pix_h.at[wid], pix_v)
        pltpu.sync_copy(valse_h.at[wid], vals_v)

        def g_cp(ci, gb, sem):
            return pltpu.make_async_copy(table_h.at[gidx_v.at[ci]], gb, sem)

        def s_cp(ci, db, sem):
            return pltpu.make_async_copy(db, f_h.at[pix_v.at[ci]], sem)

        def compose(ci, gb, db):
            for p in range(CHP):
                r0 = p * 4
                dd = p * 25
                vb = (ci * CHP + p) * 48
                vv0 = vals_v[pl.ds(vb, 16)]
                vv1 = vals_v[pl.ds(vb + 16, 16)]
                vv2 = vals_v[pl.ds(vb + 32, 16)]

                def seg_body(seg, c3, r0=r0, dd=dd, vv0=vv0, vv1=vv1, vv2=vv2):
                    col = seg * C
                    for cc in range(C // 16):
                        s = pl.ds(col + cc * 16, 16)
                        o = cc * 16
                        db[dd + seg, pl.ds(o, 16)] = gb[r0, s]
                        db[dd + seg, pl.ds(C + o, 16)] = gb[r0 + 1, s] * vv0
                        db[dd + seg, pl.ds(2 * C + o, 16)] = gb[r0 + 2, s] * vv1
                        db[dd + seg, pl.ds(3 * C + o, 16)] = gb[r0 + 3, s] * vv2
                    return c3

                lax.fori_loop(0, 25, seg_body, 0)

        g_cp(0, g0, sem_g0).start()

        def pair_body(kk, carry):
            ca = 2 * kk
            cb = 2 * kk + 1
            cp = jnp.maximum(ca - 2, 0)
            g_cp(cb, g1, sem_g1).start()
            g_cp(ca, g0, sem_g0).wait()

            @pl.when(kk > 0)
            def _():
                s_cp(cp, d0, sem_s0).wait()

            compose(ca, g0, d0)
            s_cp(ca, d0, sem_s0).start()
            @pl.when(cb + 1 < NCH)
            def _():
                g_cp(cb + 1, g0, sem_g0).start()

            g_cp(cb, g1, sem_g1).wait()

            @pl.when(kk > 0)
            def _():
                s_cp(cp + 1, d1, sem_s1).wait()

            compose(cb, g1, d1)
            s_cp(cb, d1, sem_s1).start()
            return carry

        lax.fori_loop(0, NCH // 2, pair_body, 0)
        s_cp(NCH - 2, d0, sem_s0).wait()
        s_cp(NCH - 1, d1, sem_s1).wait()

    return k(table, gidx, _PIX, valse)


def _wnorm_body(wv_ref, wg_ref, o_ref):
    w = wv_ref[...]
    n2 = jnp.sum(w * w, axis=0, keepdims=True)
    o_ref[...] = w * (wg_ref[...] * lax.rsqrt(n2))


def _slab_copy(f_ref, buf, sem, step, slot):
    b2 = step // N_BLK
    i2 = step % N_BLK
    s0 = b2 * BSTRIDE + R_BLK * i2 * WPAD
    return pltpu.make_async_copy(
        f_ref.at[pl.ds(s0, FLAT_IN)], buf.at[slot], sem.at[slot])


def _conv_body(f_ref, w_ref, b_ref, o_ref, buf, sem):
    step = pl.program_id(0) * N_BLK + pl.program_id(1)
    slot = lax.rem(step, 2)

    @pl.when(step == 0)
    def _():
        _slab_copy(f_ref, buf, sem, 0, 0).start()

    @pl.when(step + 1 < B_ * N_BLK)
    def _():
        _slab_copy(f_ref, buf, sem, step + 1, lax.rem(step + 1, 2)).start()

    _slab_copy(f_ref, buf, sem, step, slot).wait()
    acc = jnp.zeros((FLAT_OUT, C), jnp.float32)
    for dy in range(3):
        for dx in range(3):
            off = dy * WPAD + dx
            a = buf[slot, pl.ds(off, FLAT_OUT), :]
            acc = acc + jnp.dot(a, w_ref[dy, dx],
                                preferred_element_type=jnp.float32)
    o_ref[0, 0] = acc + b_ref[0]


def kernel(img, corr_index, corr_values, w_v, w_g, bias):
    f32 = jnp.float32
    xpad = jnp.pad(img, ((0, 0), (0, 0), (0, 1), (0, 1)))
    table = (xpad.reshape(B_, C, NH, P, NH, P)
             .transpose(0, 2, 4, 3, 5, 1)
             .reshape(B_ * L, D))
    table = jnp.pad(table, ((0, 1), (0, DP - D)))

    nbr = (corr_index.transpose(1, 0, 2)
           + (jnp.arange(B_, dtype=jnp.int32) * L)[:, None, None]
           ).reshape(B_ * L, 3)
    gidx = jnp.concatenate(
        [jnp.arange(B_ * L, dtype=jnp.int32)[:, None], nbr], axis=1)
    gidx = jnp.concatenate(
        [gidx, jnp.full((GP_TOTAL - B_ * L, 4), ZROW, jnp.int32)])
    gidx = gidx.reshape(NWORK, NCH, 4 * CHP)

    vals = corr_values.transpose(1, 0, 2).reshape(B_ * L, 3)
    vals = jnp.concatenate([vals, jnp.zeros((GP_TOTAL - B_ * L, 3), f32)])
    valse = jnp.broadcast_to(vals[:, :, None], (GP_TOTAL, 3, 16))
    valse = valse.reshape(NWORK, PPW * 48)

    f_flat = _sc_compose(table, gidx, valse)

    w_vt = jnp.transpose(w_v.reshape(C, 4 * C * 9), (1, 0))
    wsc = pl.pallas_call(
        _wnorm_body,
        out_shape=jax.ShapeDtypeStruct((4 * C * 9, C), f32),
    )(w_vt, w_g.reshape(1, C))
    wtaps = wsc.reshape(4 * C, 3, 3, C).transpose(1, 2, 0, 3)

    out = pl.pallas_call(
        _conv_body,
        grid=(B_, N_BLK),
        in_specs=[
            pl.BlockSpec(memory_space=pl.ANY),
            pl.BlockSpec((3, 3, 4 * C, C), lambda b, i: (0, 0, 0, 0)),
            pl.BlockSpec((1, C), lambda b, i: (0, 0)),
        ],
        out_specs=pl.BlockSpec((1, 1, FLAT_OUT, C), lambda b, i: (b, i, 0, 0)),
        out_shape=jax.ShapeDtypeStruct((B_, N_BLK, FLAT_OUT, C), f32),
        scratch_shapes=[
            pltpu.VMEM((2, FLAT_IN, 4 * C), f32),
            pltpu.SemaphoreType.DMA((2,)),
        ],
    )(f_flat, wtaps, bias.reshape(1, C))

    out = (out.reshape(B_, N_BLK, R_BLK, WPAD, C)[:, :, :, :W, :]
           .reshape(B_, H, W, C).transpose(0, 3, 1, 2))
    return out
